# Initial kernel scaffold; baseline (speedup 1.0000x reference)
import jax
import jax.numpy as jnp
from jax import lax
from jax.experimental import pallas as pl
from jax.experimental.pallas import tpu as pltpu
from jax.experimental.pallas import tpu_sc as plsc

NORM_FACTOR = 100.0

NC = 2
NS = 16
NW = NC * NS
CH = 128
BE = 512
LANE16 = 16


def _mesh():
    return plsc.VectorSubcoreMesh(core_axis_name="c", subcore_axis_name="s",
                                  num_cores=NC, num_subcores=NS)


def _silu(v):
    return v * jax.nn.sigmoid(v)


def _geom_call(xT, row1, col1, n_nodes, e_pad):
    ew = e_pad // NW

    def body(xT_h, row_h, col_h, rad_h, c0_h, c1_h, c2_h,
             row_vm, col_vm, xT_vm, rad_vm, c0_vm, c1_vm, c2_vm):
        wid = lax.axis_index("s") * NC + lax.axis_index("c")
        base = wid * ew
        pltpu.sync_copy(row_h.at[pl.ds(base, ew)], row_vm)
        pltpu.sync_copy(col_h.at[pl.ds(base, ew)], col_vm)
        pltpu.sync_copy(xT_h, xT_vm)

        def grp(g, carry):
            s = g * LANE16
            ir = row_vm[pl.ds(s, LANE16)]
            ic = col_vm[pl.ds(s, LANE16)]
            d = [plsc.load_gather(xT_vm.at[j], [ir])
                 - plsc.load_gather(xT_vm.at[j], [ic]) for j in range(3)]
            rad = d[0] * d[0] + d[1] * d[1] + d[2] * d[2]
            rad_vm[pl.ds(s, LANE16)] = rad
            a = rad + 1e-8
            bits = plsc.bitcast(a, jnp.int32)
            y = plsc.bitcast(lax.shift_right_logical(bits, 1) + 0x1FBD1DF5,
                             jnp.float32)
            y = 0.5 * (y + a / y)
            y = 0.5 * (y + a / y)
            y = 0.5 * (y + a / y)
            inv = 1.0 / (y + 1.0)
            c0_vm[pl.ds(s, LANE16)] = d[0] * inv
            c1_vm[pl.ds(s, LANE16)] = d[1] * inv
            c2_vm[pl.ds(s, LANE16)] = d[2] * inv
            return carry

        lax.fori_loop(0, ew // LANE16, grp, 0)
        pltpu.sync_copy(rad_vm, rad_h.at[pl.ds(base, ew)])
        pltpu.sync_copy(c0_vm, c0_h.at[pl.ds(base, ew)])
        pltpu.sync_copy(c1_vm, c1_h.at[pl.ds(base, ew)])
        pltpu.sync_copy(c2_vm, c2_h.at[pl.ds(base, ew)])

    f32 = jnp.float32
    fn = pl.kernel(
        body,
        out_type=[jax.ShapeDtypeStruct((e_pad,), f32)] * 4,
        mesh=_mesh(),
        scratch_types=[
            pltpu.VMEM((ew,), jnp.int32),
            pltpu.VMEM((ew,), jnp.int32),
            pltpu.VMEM((3, n_nodes), f32),
            pltpu.VMEM((ew,), f32),
            pltpu.VMEM((ew,), f32),
            pltpu.VMEM((ew,), f32),
            pltpu.VMEM((ew,), f32),
        ],
    )
    return fn(xT, row1, col1)


def _gather_call(tbl, row2, col2, e_pad):
    n_nodes, h_dim = tbl.shape
    nchunk = e_pad // (NW * CH)
    f32 = jnp.float32

    def body(tbl_h, row_h, col_h, hr_h, hc_h,
             ir_vm, ic_vm, bufr, bufc, semr, semc):
        wid = lax.axis_index("s") * NC + lax.axis_index("c")
        cb = wid * nchunk
        pltpu.sync_copy(row_h.at[pl.ds(cb, nchunk)], ir_vm)
        pltpu.sync_copy(col_h.at[pl.ds(cb, nchunk)], ic_vm)

        def chunk(k, carry):
            cr = pltpu.async_copy(tbl_h.at[ir_vm.at[k]], bufr, semr)
            cc = pltpu.async_copy(tbl_h.at[ic_vm.at[k]], bufc, semc)
            cr.wait()
            cc.wait()
            r0 = (cb + k) * CH
            pltpu.sync_copy(bufr, hr_h.at[pl.ds(r0, CH)])
            pltpu.sync_copy(bufc, hc_h.at[pl.ds(r0, CH)])
            return carry

        lax.fori_loop(0, nchunk, chunk, 0)

    fn = pl.kernel(
        body,
        out_type=[jax.ShapeDtypeStruct((e_pad, h_dim), f32)] * 2,
        mesh=_mesh(),
        scratch_types=[
            pltpu.VMEM((nchunk, CH), jnp.int32),
            pltpu.VMEM((nchunk, CH), jnp.int32),
            pltpu.VMEM((CH, h_dim), f32),
            pltpu.VMEM((CH, h_dim), f32),
            pltpu.SemaphoreType.DMA,
            pltpu.SemaphoreType.DMA,
        ],
    )
    return fn(tbl, row2, col2)


def _segsum_call(msg, row2, n_nodes):
    e_pad, h_dim = msg.shape
    nchunk = e_pad // (NW * CH)
    zr = n_nodes // NS
    f32 = jnp.float32
    zeros = jnp.zeros((zr, h_dim), f32)

    def body(msg_h, row_h, zero_h, part_h, idx_vm, mbuf, acc):
        cid = lax.axis_index("c")
        sid = lax.axis_index("s")
        wid = sid * NC + cid
        cb = wid * nchunk
        pltpu.sync_copy(row_h.at[pl.ds(cb, nchunk)], idx_vm)
        pltpu.sync_copy(zero_h, acc.at[pl.ds(sid * zr, zr)])
        plsc.subcore_barrier()

        def chunk(k, carry):
            pltpu.sync_copy(msg_h.at[pl.ds((cb + k) * CH, CH)], mbuf)
            pltpu.sync_copy(mbuf, acc.at[idx_vm.at[k]], add=True)
            return carry

        lax.fori_loop(0, nchunk, chunk, 0)
        plsc.subcore_barrier()
        pltpu.sync_copy(acc.at[pl.ds(sid * zr, zr)],
                        part_h.at[cid, pl.ds(sid * zr, zr)])

    fn = pl.kernel(
        body,
        out_type=jax.ShapeDtypeStruct((NC, n_nodes, h_dim), f32),
        mesh=_mesh(),
        scratch_types=[
            pltpu.VMEM((nchunk, CH), jnp.int32),
            pltpu.VMEM((CH, h_dim), f32),
            pltpu.VMEM_SHARED((n_nodes, h_dim), f32),
        ],
    )
    return fn(msg, row2, zeros)


def _segsum3_call(phi, c0, c1, c2, row2, n_nodes):
    e_pad = phi.shape[0]
    ew = e_pad // NW
    nchunk = ew // CH
    zr = n_nodes // NS
    f32 = jnp.float32
    zeros = jnp.zeros((zr, LANE16), f32)

    def body(phi_h, c0_h, c1_h, c2_h, row_h, zero_h, px_h,
             idx_vm, phi_vm, c0_vm, c1_vm, c2_vm, tr, acc):
        cid = lax.axis_index("c")
        sid = lax.axis_index("s")
        wid = sid * NC + cid
        cb = wid * nchunk
        base = wid * ew
        pltpu.sync_copy(row_h.at[pl.ds(cb, nchunk)], idx_vm)
        pltpu.sync_copy(phi_h.at[pl.ds(base, ew)], phi_vm)
        pltpu.sync_copy(c0_h.at[pl.ds(base, ew)], c0_vm)
        pltpu.sync_copy(c1_h.at[pl.ds(base, ew)], c1_vm)
        pltpu.sync_copy(c2_h.at[pl.ds(base, ew)], c2_vm)

        def zb(r, carry):
            tr[r, :] = jnp.zeros((LANE16,), f32)
            return carry

        lax.fori_loop(0, CH, zb, 0)
        pltpu.sync_copy(zero_h, acc.at[pl.ds(sid * zr, zr)])
        plsc.subcore_barrier()
        iot = lax.iota(jnp.int32, LANE16)

        def chunk(k, carry):
            def grp(g, inner):
                e0 = k * CH + g * LANE16
                p = phi_vm[pl.ds(e0, LANE16)]
                lvec = g * LANE16 + iot
                for j, cv in enumerate((c0_vm, c1_vm, c2_vm)):
                    vals = cv[pl.ds(e0, LANE16)] * p
                    plsc.store_scatter(
                        tr, [lvec, jnp.full((LANE16,), j, jnp.int32)], vals)
                return inner

            lax.fori_loop(0, CH // LANE16, grp, 0)
            pltpu.sync_copy(tr, acc.at[idx_vm.at[k]], add=True)
            return carry

        lax.fori_loop(0, nchunk, chunk, 0)
        plsc.subcore_barrier()
        pltpu.sync_copy(acc.at[pl.ds(sid * zr, zr)],
                        px_h.at[cid, pl.ds(sid * zr, zr)])

    fn = pl.kernel(
        body,
        out_type=jax.ShapeDtypeStruct((NC, n_nodes, LANE16), f32),
        mesh=_mesh(),
        scratch_types=[
            pltpu.VMEM((nchunk, CH), jnp.int32),
            pltpu.VMEM((ew,), f32),
            pltpu.VMEM((ew,), f32),
            pltpu.VMEM((ew,), f32),
            pltpu.VMEM((ew,), f32),
            pltpu.VMEM((CH, LANE16), f32),
            pltpu.VMEM_SHARED((n_nodes, LANE16), f32),
        ],
    )
    return fn(phi, c0, c1, c2, row2, zeros)


def _edge_mlp_call(hr, hc, rad3, ea3, em3, w1a, w1b, wr, wa, b1, w2, b2,
                   last_t, last_b, attention):
    e_pad, h_dim = hr.shape
    g = e_pad // BE
    f32 = jnp.float32

    def body(hr_ref, hc_ref, rad_ref, ea_ref, em_ref, w1a_ref, w1b_ref,
             wr_ref, wa_ref, b1_ref, w2_ref, b2_ref, lt_ref, lb_ref, out_ref):
        z = (jnp.dot(hr_ref[...], w1a_ref[...], preferred_element_type=f32)
             + jnp.dot(hc_ref[...], w1b_ref[...], preferred_element_type=f32))
        z = (z + rad_ref[0, 0, :][:, None] * wr_ref[...]
             + ea_ref[0, 0, :][:, None] * wa_ref[...] + b1_ref[...])
        m = _silu(z)
        m = _silu(jnp.dot(m, w2_ref[...], preferred_element_type=f32)
                  + b2_ref[...])
        em = em_ref[0, 0, :]
        if attention:
            att = jax.nn.sigmoid(
                jnp.sum(m * lt_ref[...], axis=1, keepdims=True) + lb_ref[0, 0])
            out_ref[...] = m * att * em[:, None]
        else:
            out_ref[0, 0, :] = jnp.sum(m * lt_ref[...], axis=1) * em

    full = lambda shp: pl.BlockSpec(shp, lambda i: tuple(0 for _ in shp))
    in_specs = [
        pl.BlockSpec((BE, h_dim), lambda i: (i, 0)),
        pl.BlockSpec((BE, h_dim), lambda i: (i, 0)),
        pl.BlockSpec((1, 1, BE), lambda i: (i, 0, 0)),
        pl.BlockSpec((1, 1, BE), lambda i: (i, 0, 0)),
        pl.BlockSpec((1, 1, BE), lambda i: (i, 0, 0)),
        full((h_dim, h_dim)), full((h_dim, h_dim)),
        full((1, h_dim)), full((1, h_dim)), full((1, h_dim)),
        full((h_dim, h_dim)), full((1, h_dim)),
        full((1, h_dim)), full((1, 1)),
    ]
    if attention:
        out_specs = pl.BlockSpec((BE, h_dim), lambda i: (i, 0))
        out_shape = jax.ShapeDtypeStruct((e_pad, h_dim), f32)
    else:
        out_specs = pl.BlockSpec((1, 1, BE), lambda i: (i, 0, 0))
        out_shape = jax.ShapeDtypeStruct((g, 1, BE), f32)
    return pl.pallas_call(
        body, grid=(g,), in_specs=in_specs, out_specs=out_specs,
        out_shape=out_shape,
    )(hr, hc, rad3, ea3, em3, w1a, w1b, wr, wa, b1, w2, b2, last_t, last_b)


def _node_mlp_call(h, part, nm3, w1a, w1b, b1, w2, b2):
    n_nodes, h_dim = h.shape
    bn = 500
    g = n_nodes // bn
    f32 = jnp.float32

    def body(h_ref, p_ref, nm_ref, w1a_ref, w1b_ref, b1_ref, w2_ref, b2_ref,
             out_ref):
        hv = h_ref[...]
        agg = (p_ref[0] + p_ref[1]) * (1.0 / NORM_FACTOR)
        z = (jnp.dot(hv, w1a_ref[...], preferred_element_type=f32)
             + jnp.dot(agg, w1b_ref[...], preferred_element_type=f32)
             + b1_ref[...])
        o = jnp.dot(_silu(z), w2_ref[...], preferred_element_type=f32) \
            + b2_ref[...]
        out_ref[...] = (hv + o) * nm_ref[0, 0, :][:, None]

    full = lambda shp: pl.BlockSpec(shp, lambda i: tuple(0 for _ in shp))
    return pl.pallas_call(
        body, grid=(g,),
        in_specs=[
            pl.BlockSpec((bn, h_dim), lambda i: (i, 0)),
            pl.BlockSpec((NC, bn, h_dim), lambda i: (0, i, 0)),
            pl.BlockSpec((1, 1, bn), lambda i: (i, 0, 0)),
            full((h_dim, h_dim)), full((h_dim, h_dim)), full((1, h_dim)),
            full((h_dim, h_dim)), full((1, h_dim)),
        ],
        out_specs=pl.BlockSpec((bn, h_dim), lambda i: (i, 0)),
        out_shape=jax.ShapeDtypeStruct((n_nodes, h_dim), f32),
    )(h, part, nm3, w1a, w1b, b1, w2, b2)


def _xupd_call(x16, px, nm3):
    n_nodes = x16.shape[0]
    bn = 500
    g = n_nodes // bn
    f32 = jnp.float32

    def body(x_ref, px_ref, nm_ref, out_ref):
        agg = (px_ref[0] + px_ref[1]) * (1.0 / NORM_FACTOR)
        out_ref[...] = (x_ref[...] + agg) * nm_ref[0, 0, :][:, None]

    return pl.pallas_call(
        body, grid=(g,),
        in_specs=[
            pl.BlockSpec((bn, LANE16), lambda i: (i, 0)),
            pl.BlockSpec((NC, bn, LANE16), lambda i: (0, i, 0)),
            pl.BlockSpec((1, 1, bn), lambda i: (i, 0, 0)),
        ],
        out_specs=pl.BlockSpec((bn, LANE16), lambda i: (i, 0)),
        out_shape=jax.ShapeDtypeStruct((n_nodes, LANE16), f32),
    )(x16, px, nm3)


def kernel(h, x, edge_index, batch_size, node_mask, edge_mask, edge_attr,
           params):
    n_nodes, h_dim = h.shape
    e = edge_index.shape[1]
    quant = NW * CH
    e_pad = ((e + quant - 1) // quant) * quant
    pad = e_pad - e
    f32 = jnp.float32

    row1 = jnp.concatenate([edge_index[0], jnp.zeros((pad,), jnp.int32)])
    col1 = jnp.concatenate([edge_index[1], jnp.zeros((pad,), jnp.int32)])
    row2 = row1.reshape(-1, CH)
    col2 = col1.reshape(-1, CH)
    ea1 = jnp.concatenate([edge_attr[:, 0], jnp.zeros((pad,), f32)])
    em1 = jnp.concatenate([edge_mask[:, 0], jnp.zeros((pad,), f32)])

    ge = e_pad // BE
    ea3 = ea1.reshape(ge, 1, BE)
    em3 = em1.reshape(ge, 1, BE)
    gn = n_nodes // 500
    nm3 = node_mask[:, 0].reshape(gn, 1, 500)

    rad1, c0, c1, c2 = _geom_call(x.T, row1, col1, n_nodes, e_pad)
    rad3 = rad1.reshape(ge, 1, BE)

    hcur = h
    for i in range(2):
        p = params['gcl%d' % i]
        w1a, w1b = p['eW1'][:h_dim], p['eW1'][h_dim:2 * h_dim]
        wr = p['eW1'][2 * h_dim:2 * h_dim + 1]
        wa = p['eW1'][2 * h_dim + 1:2 * h_dim + 2]
        hr, hc = _gather_call(hcur, row2, col2, e_pad)
        msg = _edge_mlp_call(hr, hc, rad3, ea3, em3, w1a, w1b, wr, wa,
                             p['eb1'].reshape(1, h_dim), p['eW2'],
                             p['eb2'].reshape(1, h_dim),
                             p['aW'].reshape(1, h_dim),
                             p['ab'].reshape(1, 1), attention=True)
        part = _segsum_call(msg, row2, n_nodes)
        hcur = _node_mlp_call(hcur, part, nm3,
                              p['nW1'][:h_dim], p['nW1'][h_dim:],
                              p['nb1'].reshape(1, h_dim), p['nW2'],
                              p['nb2'].reshape(1, h_dim))

    p = params['equiv']
    w1a, w1b = p['cW1'][:h_dim], p['cW1'][h_dim:2 * h_dim]
    wr = p['cW1'][2 * h_dim:2 * h_dim + 1]
    wa = p['cW1'][2 * h_dim + 1:2 * h_dim + 2]
    hr, hc = _gather_call(hcur, row2, col2, e_pad)
    phi3 = _edge_mlp_call(hr, hc, rad3, ea3, em3, w1a, w1b, wr, wa,
                          p['cb1'].reshape(1, h_dim), p['cW2'],
                          p['cb2'].reshape(1, h_dim),
                          p['cW3'].reshape(1, h_dim),
                          jnp.zeros((1, 1), f32), attention=False)
    phi1 = phi3.reshape(e_pad)
    px = _segsum3_call(phi1, c0, c1, c2, row2, n_nodes)

    x16 = jnp.pad(x, ((0, 0), (0, LANE16 - x.shape[1])))
    xo16 = _xupd_call(x16, px, nm3)
    x_new = xo16[:, :x.shape[1]]
    return hcur, x_new


# speedup vs baseline: 1.4742x; 1.4742x over previous
import jax
import jax.numpy as jnp
from jax import lax
from jax.experimental import pallas as pl
from jax.experimental.pallas import tpu as pltpu
from jax.experimental.pallas import tpu_sc as plsc

NORM_FACTOR = 100.0

NC = 2
NS = 16
NW = NC * NS
CH = 128
BE = 512
LANE16 = 16


def _mesh():
    return plsc.VectorSubcoreMesh(core_axis_name="c", subcore_axis_name="s",
                                  num_cores=NC, num_subcores=NS)


_SC_PARAMS = pltpu.CompilerParams(needs_layout_passes=False)


def _silu(v):
    return v * jax.nn.sigmoid(v)


def _geom_call(x0, x1, x2, row1, col1, n_nodes, e_pad):
    ew = e_pad // NW

    def body(x0_h, x1_h, x2_h, row_h, col_h, rad_h, c0_h, c1_h, c2_h,
             row_vm, col_vm, x0_vm, x1_vm, x2_vm, rad_vm, c0_vm, c1_vm, c2_vm):
        wid = lax.axis_index("s") * NC + lax.axis_index("c")
        base = wid * ew
        pltpu.sync_copy(row_h.at[pl.ds(base, ew)], row_vm)
        pltpu.sync_copy(col_h.at[pl.ds(base, ew)], col_vm)
        pltpu.sync_copy(x0_h, x0_vm)
        pltpu.sync_copy(x1_h, x1_vm)
        pltpu.sync_copy(x2_h, x2_vm)

        def grp(g, carry):
            s = g * LANE16
            ir = row_vm[pl.ds(s, LANE16)]
            ic = col_vm[pl.ds(s, LANE16)]
            d = [plsc.load_gather(xv, [ir])
                 - plsc.load_gather(xv, [ic]) for xv in (x0_vm, x1_vm, x2_vm)]
            rad = d[0] * d[0] + d[1] * d[1] + d[2] * d[2]
            rad_vm[pl.ds(s, LANE16)] = rad
            a = rad + 1e-8
            bits = plsc.bitcast(a, jnp.int32)
            y = plsc.bitcast(lax.shift_right_logical(bits, 1) + 0x1FBD1DF5,
                             jnp.float32)
            y = 0.5 * (y + a / y)
            y = 0.5 * (y + a / y)
            y = 0.5 * (y + a / y)
            inv = 1.0 / (y + 1.0)
            c0_vm[pl.ds(s, LANE16)] = d[0] * inv
            c1_vm[pl.ds(s, LANE16)] = d[1] * inv
            c2_vm[pl.ds(s, LANE16)] = d[2] * inv
            return carry

        lax.fori_loop(0, ew // LANE16, grp, 0)
        pltpu.sync_copy(rad_vm, rad_h.at[pl.ds(base, ew)])
        pltpu.sync_copy(c0_vm, c0_h.at[pl.ds(base, ew)])
        pltpu.sync_copy(c1_vm, c1_h.at[pl.ds(base, ew)])
        pltpu.sync_copy(c2_vm, c2_h.at[pl.ds(base, ew)])

    f32 = jnp.float32
    fn = pl.kernel(
        body,
        out_type=[jax.ShapeDtypeStruct((e_pad,), f32)] * 4,
        mesh=_mesh(),
        compiler_params=_SC_PARAMS,
        scratch_types=[
            pltpu.VMEM((ew,), jnp.int32),
            pltpu.VMEM((ew,), jnp.int32),
            pltpu.VMEM((n_nodes,), f32),
            pltpu.VMEM((n_nodes,), f32),
            pltpu.VMEM((n_nodes,), f32),
            pltpu.VMEM((ew,), f32),
            pltpu.VMEM((ew,), f32),
            pltpu.VMEM((ew,), f32),
            pltpu.VMEM((ew,), f32),
        ],
    )
    return fn(x0, x1, x2, row1, col1)


def _gather_call(tbl, row2, col2, e_pad):
    n_nodes, h_dim = tbl.shape
    nchunk = e_pad // (NW * CH)
    f32 = jnp.float32

    def body(tbl_h, row_h, col_h, hr_h, hc_h,
             ir_vm, ic_vm, bufr, bufc, semr, semc):
        wid = lax.axis_index("s") * NC + lax.axis_index("c")
        cb = wid * nchunk
        pltpu.sync_copy(row_h.at[pl.ds(cb, nchunk)], ir_vm)
        pltpu.sync_copy(col_h.at[pl.ds(cb, nchunk)], ic_vm)

        def chunk(k, carry):
            cr = pltpu.async_copy(tbl_h.at[ir_vm.at[k]], bufr, semr)
            cc = pltpu.async_copy(tbl_h.at[ic_vm.at[k]], bufc, semc)
            cr.wait()
            cc.wait()
            r0 = (cb + k) * CH
            pltpu.sync_copy(bufr, hr_h.at[pl.ds(r0, CH)])
            pltpu.sync_copy(bufc, hc_h.at[pl.ds(r0, CH)])
            return carry

        lax.fori_loop(0, nchunk, chunk, 0)

    fn = pl.kernel(
        body,
        out_type=[jax.ShapeDtypeStruct((e_pad, h_dim), f32)] * 2,
        mesh=_mesh(),
        compiler_params=_SC_PARAMS,
        scratch_types=[
            pltpu.VMEM((nchunk, CH), jnp.int32),
            pltpu.VMEM((nchunk, CH), jnp.int32),
            pltpu.VMEM((CH, h_dim), f32),
            pltpu.VMEM((CH, h_dim), f32),
            pltpu.SemaphoreType.DMA,
            pltpu.SemaphoreType.DMA,
        ],
    )
    return fn(tbl, row2, col2)


def _segsum_call(msg, row2, n_nodes):
    e_pad, h_dim = msg.shape
    nchunk = e_pad // (NW * CH)
    zr = n_nodes // NS
    f32 = jnp.float32
    zeros = jnp.zeros((zr, h_dim), f32)

    def body(msg_h, row_h, zero_h, part_h, idx_vm, mbuf, acc):
        cid = lax.axis_index("c")
        sid = lax.axis_index("s")
        wid = sid * NC + cid
        cb = wid * nchunk
        pltpu.sync_copy(row_h.at[pl.ds(cb, nchunk)], idx_vm)
        pltpu.sync_copy(zero_h, acc.at[pl.ds(sid * zr, zr)])
        plsc.subcore_barrier()

        def chunk(k, carry):
            pltpu.sync_copy(msg_h.at[pl.ds((cb + k) * CH, CH)], mbuf)
            pltpu.sync_copy(mbuf, acc.at[idx_vm.at[k]], add=True)
            return carry

        lax.fori_loop(0, nchunk, chunk, 0)
        plsc.subcore_barrier()
        pltpu.sync_copy(acc.at[pl.ds(sid * zr, zr)],
                        part_h.at[cid, pl.ds(sid * zr, zr)])

    fn = pl.kernel(
        body,
        out_type=jax.ShapeDtypeStruct((NC, n_nodes, h_dim), f32),
        mesh=_mesh(),
        compiler_params=_SC_PARAMS,
        scratch_types=[
            pltpu.VMEM((nchunk, CH), jnp.int32),
            pltpu.VMEM((CH, h_dim), f32),
            pltpu.VMEM_SHARED((n_nodes, h_dim), f32),
        ],
    )
    return fn(msg, row2, zeros)


def _edge_mlp_call(hr, hc, rad3, ea3, em3, w1a, w1b, wr, wa, b1, w2, b2,
                   last_t, last_b, attention):
    e_pad, h_dim = hr.shape
    g = e_pad // BE
    f32 = jnp.float32

    def body(hr_ref, hc_ref, rad_ref, ea_ref, em_ref, w1a_ref, w1b_ref,
             wr_ref, wa_ref, b1_ref, w2_ref, b2_ref, lt_ref, lb_ref, out_ref):
        z = (jnp.dot(hr_ref[...], w1a_ref[...], preferred_element_type=f32)
             + jnp.dot(hc_ref[...], w1b_ref[...], preferred_element_type=f32))
        z = (z + rad_ref[0, 0, :][:, None] * wr_ref[...]
             + ea_ref[0, 0, :][:, None] * wa_ref[...] + b1_ref[...])
        m = _silu(z)
        m = _silu(jnp.dot(m, w2_ref[...], preferred_element_type=f32)
                  + b2_ref[...])
        em = em_ref[0, 0, :]
        if attention:
            att = jax.nn.sigmoid(
                jnp.sum(m * lt_ref[...], axis=1, keepdims=True) + lb_ref[0, 0])
            out_ref[...] = m * att * em[:, None]
        else:
            out_ref[0, 0, :] = jnp.sum(m * lt_ref[...], axis=1) * em

    full = lambda shp: pl.BlockSpec(shp, lambda i: tuple(0 for _ in shp))
    in_specs = [
        pl.BlockSpec((BE, h_dim), lambda i: (i, 0)),
        pl.BlockSpec((BE, h_dim), lambda i: (i, 0)),
        pl.BlockSpec((1, 1, BE), lambda i: (i, 0, 0)),
        pl.BlockSpec((1, 1, BE), lambda i: (i, 0, 0)),
        pl.BlockSpec((1, 1, BE), lambda i: (i, 0, 0)),
        full((h_dim, h_dim)), full((h_dim, h_dim)),
        full((1, h_dim)), full((1, h_dim)), full((1, h_dim)),
        full((h_dim, h_dim)), full((1, h_dim)),
        full((1, h_dim)), full((1, 1)),
    ]
    if attention:
        out_specs = pl.BlockSpec((BE, h_dim), lambda i: (i, 0))
        out_shape = jax.ShapeDtypeStruct((e_pad, h_dim), f32)
    else:
        out_specs = pl.BlockSpec((1, 1, BE), lambda i: (i, 0, 0))
        out_shape = jax.ShapeDtypeStruct((g, 1, BE), f32)
    return pl.pallas_call(
        body, grid=(g,), in_specs=in_specs, out_specs=out_specs,
        out_shape=out_shape,
    )(hr, hc, rad3, ea3, em3, w1a, w1b, wr, wa, b1, w2, b2, last_t, last_b)


def _edge_equiv_call(hr, hc, rad3, ea3, em3, c03, c13, c23,
                     w1a, w1b, wr, wa, b1, w2, b2, w3t):
    e_pad, h_dim = hr.shape
    g = e_pad // BE
    f32 = jnp.float32

    def body(hr_ref, hc_ref, rad_ref, ea_ref, em_ref, c0_ref, c1_ref, c2_ref,
             w1a_ref, w1b_ref, wr_ref, wa_ref, b1_ref, w2_ref, b2_ref,
             w3_ref, out_ref):
        z = (jnp.dot(hr_ref[...], w1a_ref[...], preferred_element_type=f32)
             + jnp.dot(hc_ref[...], w1b_ref[...], preferred_element_type=f32))
        z = (z + rad_ref[0, 0, :][:, None] * wr_ref[...]
             + ea_ref[0, 0, :][:, None] * wa_ref[...] + b1_ref[...])
        m = _silu(z)
        m = _silu(jnp.dot(m, w2_ref[...], preferred_element_type=f32)
                  + b2_ref[...])
        phi = jnp.sum(m * w3_ref[...], axis=1) * em_ref[0, 0, :]
        cols = [(phi * c_ref[0, 0, :])[:, None]
                for c_ref in (c0_ref, c1_ref, c2_ref)]
        out_ref[...] = jnp.concatenate(
            cols + [jnp.zeros((BE, h_dim - 3), f32)], axis=1)

    full = lambda shp: pl.BlockSpec(shp, lambda i: tuple(0 for _ in shp))
    edge1 = lambda: pl.BlockSpec((1, 1, BE), lambda i: (i, 0, 0))
    in_specs = [
        pl.BlockSpec((BE, h_dim), lambda i: (i, 0)),
        pl.BlockSpec((BE, h_dim), lambda i: (i, 0)),
        edge1(), edge1(), edge1(), edge1(), edge1(), edge1(),
        full((h_dim, h_dim)), full((h_dim, h_dim)),
        full((1, h_dim)), full((1, h_dim)), full((1, h_dim)),
        full((h_dim, h_dim)), full((1, h_dim)), full((1, h_dim)),
    ]
    return pl.pallas_call(
        body, grid=(g,), in_specs=in_specs,
        out_specs=pl.BlockSpec((BE, h_dim), lambda i: (i, 0)),
        out_shape=jax.ShapeDtypeStruct((e_pad, h_dim), f32),
    )(hr, hc, rad3, ea3, em3, c03, c13, c23,
      w1a, w1b, wr, wa, b1, w2, b2, w3t)


def _node_mlp_call(h, part, nm3, w1a, w1b, b1, w2, b2):
    n_nodes, h_dim = h.shape
    bn = 512
    g = n_nodes // bn
    f32 = jnp.float32

    def body(h_ref, p_ref, nm_ref, w1a_ref, w1b_ref, b1_ref, w2_ref, b2_ref,
             out_ref):
        hv = h_ref[...]
        agg = (p_ref[0] + p_ref[1]) * (1.0 / NORM_FACTOR)
        z = (jnp.dot(hv, w1a_ref[...], preferred_element_type=f32)
             + jnp.dot(agg, w1b_ref[...], preferred_element_type=f32)
             + b1_ref[...])
        o = jnp.dot(_silu(z), w2_ref[...], preferred_element_type=f32) \
            + b2_ref[...]
        out_ref[...] = (hv + o) * nm_ref[0, 0, :][:, None]

    full = lambda shp: pl.BlockSpec(shp, lambda i: tuple(0 for _ in shp))
    return pl.pallas_call(
        body, grid=(g,),
        in_specs=[
            pl.BlockSpec((bn, h_dim), lambda i: (i, 0)),
            pl.BlockSpec((NC, bn, h_dim), lambda i: (0, i, 0)),
            pl.BlockSpec((1, 1, bn), lambda i: (i, 0, 0)),
            full((h_dim, h_dim)), full((h_dim, h_dim)), full((1, h_dim)),
            full((h_dim, h_dim)), full((1, h_dim)),
        ],
        out_specs=pl.BlockSpec((bn, h_dim), lambda i: (i, 0)),
        out_shape=jax.ShapeDtypeStruct((n_nodes, h_dim), f32),
    )(h, part, nm3, w1a, w1b, b1, w2, b2)


def _xupd_call(x16, px, nm3):
    n_nodes = x16.shape[0]
    hp = px.shape[2]
    bn = 512
    g = n_nodes // bn
    f32 = jnp.float32

    def body(x_ref, px_ref, nm_ref, out_ref):
        agg = (px_ref[0, :, :LANE16] + px_ref[1, :, :LANE16]) \
            * (1.0 / NORM_FACTOR)
        out_ref[...] = (x_ref[...] + agg) * nm_ref[0, 0, :][:, None]

    return pl.pallas_call(
        body, grid=(g,),
        in_specs=[
            pl.BlockSpec((bn, LANE16), lambda i: (i, 0)),
            pl.BlockSpec((NC, bn, hp), lambda i: (0, i, 0)),
            pl.BlockSpec((1, 1, bn), lambda i: (i, 0, 0)),
        ],
        out_specs=pl.BlockSpec((bn, LANE16), lambda i: (i, 0)),
        out_shape=jax.ShapeDtypeStruct((n_nodes, LANE16), f32),
    )(x16, px, nm3)


def kernel(h, x, edge_index, batch_size, node_mask, edge_mask, edge_attr,
           params):
    n_nodes, h_dim = h.shape
    e = edge_index.shape[1]
    quant = NW * CH * 8
    e_pad = ((e + quant - 1) // quant) * quant
    pad = e_pad - e
    n_pad = ((n_nodes + 511) // 512) * 512
    npad_rows = n_pad - n_nodes
    f32 = jnp.float32

    row1 = jnp.concatenate([edge_index[0], jnp.zeros((pad,), jnp.int32)])
    col1 = jnp.concatenate([edge_index[1], jnp.zeros((pad,), jnp.int32)])
    row2 = row1.reshape(-1, CH)
    col2 = col1.reshape(-1, CH)
    ea1 = jnp.concatenate([edge_attr[:, 0], jnp.zeros((pad,), f32)])
    em1 = jnp.concatenate([edge_mask[:, 0], jnp.zeros((pad,), f32)])

    ge = e_pad // BE
    ea3 = ea1.reshape(ge, 1, BE)
    em3 = em1.reshape(ge, 1, BE)
    nm1 = jnp.concatenate([node_mask[:, 0], jnp.zeros((npad_rows,), f32)])
    nm3 = nm1.reshape(n_pad // 512, 1, 512)
    xp = jnp.concatenate([x, jnp.zeros((npad_rows, x.shape[1]), f32)])

    rad1, c0, c1, c2 = _geom_call(xp[:, 0], xp[:, 1], xp[:, 2], row1, col1, n_pad, e_pad)
    rad3 = rad1.reshape(ge, 1, BE)

    hcur = jnp.concatenate([h, jnp.zeros((npad_rows, h_dim), f32)])
    for i in range(2):
        p = params['gcl%d' % i]
        w1a, w1b = p['eW1'][:h_dim], p['eW1'][h_dim:2 * h_dim]
        wr = p['eW1'][2 * h_dim:2 * h_dim + 1]
        wa = p['eW1'][2 * h_dim + 1:2 * h_dim + 2]
        hr, hc = _gather_call(hcur, row2, col2, e_pad)
        msg = _edge_mlp_call(hr, hc, rad3, ea3, em3, w1a, w1b, wr, wa,
                             p['eb1'].reshape(1, h_dim), p['eW2'],
                             p['eb2'].reshape(1, h_dim),
                             p['aW'].reshape(1, h_dim),
                             p['ab'].reshape(1, 1), attention=True)
        part = _segsum_call(msg, row2, n_pad)
        hcur = _node_mlp_call(hcur, part, nm3,
                              p['nW1'][:h_dim], p['nW1'][h_dim:],
                              p['nb1'].reshape(1, h_dim), p['nW2'],
                              p['nb2'].reshape(1, h_dim))

    p = params['equiv']
    w1a, w1b = p['cW1'][:h_dim], p['cW1'][h_dim:2 * h_dim]
    wr = p['cW1'][2 * h_dim:2 * h_dim + 1]
    wa = p['cW1'][2 * h_dim + 1:2 * h_dim + 2]
    hr, hc = _gather_call(hcur, row2, col2, e_pad)
    c03 = c0.reshape(ge, 1, BE)
    c13 = c1.reshape(ge, 1, BE)
    c23 = c2.reshape(ge, 1, BE)
    trans = _edge_equiv_call(hr, hc, rad3, ea3, em3, c03, c13, c23,
                               w1a, w1b, wr, wa,
                               p['cb1'].reshape(1, h_dim), p['cW2'],
                               p['cb2'].reshape(1, h_dim),
                               p['cW3'].reshape(1, h_dim))
    px = _segsum_call(trans, row2, n_pad)

    x16 = jnp.pad(xp, ((0, 0), (0, LANE16 - x.shape[1])))
    xo16 = _xupd_call(x16, px, nm3)
    x_new = xo16[:n_nodes, :x.shape[1]]
    return hcur[:n_nodes], x_new


# speedup vs baseline: 1.6292x; 1.1052x over previous
import jax
import jax.numpy as jnp
from jax import lax
from jax.experimental import pallas as pl
from jax.experimental.pallas import tpu as pltpu
from jax.experimental.pallas import tpu_sc as plsc

NORM_FACTOR = 100.0

NC = 2
NS = 16
NW = NC * NS
CH = 128
BE = 512
LANE16 = 16


def _mesh():
    return plsc.VectorSubcoreMesh(core_axis_name="c", subcore_axis_name="s",
                                  num_cores=NC, num_subcores=NS)


_SC_PARAMS = pltpu.CompilerParams(needs_layout_passes=False)


def _silu(v):
    return v * jax.nn.sigmoid(v)


def _geom_call(x0, x1, x2, row1, col1, n_nodes, e_pad):
    ew = e_pad // NW

    def body(x0_h, x1_h, x2_h, row_h, col_h, rad_h, c0_h, c1_h, c2_h,
             row_vm, col_vm, x0_vm, x1_vm, x2_vm, rad_vm, c0_vm, c1_vm, c2_vm):
        wid = lax.axis_index("s") * NC + lax.axis_index("c")
        base = wid * ew
        pltpu.sync_copy(row_h.at[pl.ds(base, ew)], row_vm)
        pltpu.sync_copy(col_h.at[pl.ds(base, ew)], col_vm)
        pltpu.sync_copy(x0_h, x0_vm)
        pltpu.sync_copy(x1_h, x1_vm)
        pltpu.sync_copy(x2_h, x2_vm)

        def grp(g, carry):
            s = g * LANE16
            ir = row_vm[pl.ds(s, LANE16)]
            ic = col_vm[pl.ds(s, LANE16)]
            d = [plsc.load_gather(xv, [ir])
                 - plsc.load_gather(xv, [ic]) for xv in (x0_vm, x1_vm, x2_vm)]
            rad = d[0] * d[0] + d[1] * d[1] + d[2] * d[2]
            rad_vm[pl.ds(s, LANE16)] = rad
            a = rad + 1e-8
            bits = plsc.bitcast(a, jnp.int32)
            y = plsc.bitcast(lax.shift_right_logical(bits, 1) + 0x1FBD1DF5,
                             jnp.float32)
            y = 0.5 * (y + a / y)
            y = 0.5 * (y + a / y)
            y = 0.5 * (y + a / y)
            inv = 1.0 / (y + 1.0)
            c0_vm[pl.ds(s, LANE16)] = d[0] * inv
            c1_vm[pl.ds(s, LANE16)] = d[1] * inv
            c2_vm[pl.ds(s, LANE16)] = d[2] * inv
            return carry

        lax.fori_loop(0, ew // LANE16, grp, 0)
        pltpu.sync_copy(rad_vm, rad_h.at[pl.ds(base, ew)])
        pltpu.sync_copy(c0_vm, c0_h.at[pl.ds(base, ew)])
        pltpu.sync_copy(c1_vm, c1_h.at[pl.ds(base, ew)])
        pltpu.sync_copy(c2_vm, c2_h.at[pl.ds(base, ew)])

    f32 = jnp.float32
    fn = pl.kernel(
        body,
        out_type=[jax.ShapeDtypeStruct((e_pad,), f32)] * 4,
        mesh=_mesh(),
        compiler_params=_SC_PARAMS,
        scratch_types=[
            pltpu.VMEM((ew,), jnp.int32),
            pltpu.VMEM((ew,), jnp.int32),
            pltpu.VMEM((n_nodes,), f32),
            pltpu.VMEM((n_nodes,), f32),
            pltpu.VMEM((n_nodes,), f32),
            pltpu.VMEM((ew,), f32),
            pltpu.VMEM((ew,), f32),
            pltpu.VMEM((ew,), f32),
            pltpu.VMEM((ew,), f32),
        ],
    )
    return fn(x0, x1, x2, row1, col1)


def _gather_call(tbl, row2, col2, e_pad):
    n_nodes, h_dim = tbl.shape
    nchunk = e_pad // (NW * CH)
    nbuf = 2
    f32 = jnp.float32

    def body(tbl_h, row_h, col_h, hr_h, hc_h, ir_vm, ic_vm, *bufs):
        bufr = bufs[0:nbuf]
        bufc = bufs[nbuf:2 * nbuf]
        gr = bufs[2 * nbuf:3 * nbuf]
        gc = bufs[3 * nbuf:4 * nbuf]
        sr = bufs[4 * nbuf:5 * nbuf]
        sc = bufs[5 * nbuf:6 * nbuf]
        wid = lax.axis_index("s") * NC + lax.axis_index("c")
        cb = wid * nchunk
        pltpu.sync_copy(row_h.at[pl.ds(cb, nchunk)], ir_vm)
        pltpu.sync_copy(col_h.at[pl.ds(cb, nchunk)], ic_vm)

        for b in range(nbuf):
            pltpu.async_copy(tbl_h.at[ir_vm.at[b]], bufr[b], gr[b])
            pltpu.async_copy(tbl_h.at[ic_vm.at[b]], bufc[b], gc[b])

        def outer(i, carry):
            k0 = i * nbuf
            for b in range(nbuf):
                k = k0 + b
                pltpu.make_async_copy(tbl_h.at[ir_vm.at[k]], bufr[b],
                                      gr[b]).wait()
                pltpu.make_async_copy(tbl_h.at[ic_vm.at[k]], bufc[b],
                                      gc[b]).wait()
                r0 = (cb + k) * CH
                pltpu.async_copy(bufr[b], hr_h.at[pl.ds(r0, CH)], sr[b])
                pltpu.async_copy(bufc[b], hc_h.at[pl.ds(r0, CH)], sc[b])
                pltpu.make_async_copy(bufr[b], hr_h.at[pl.ds(r0, CH)],
                                      sr[b]).wait()
                pltpu.make_async_copy(bufc[b], hc_h.at[pl.ds(r0, CH)],
                                      sc[b]).wait()
                nk = k + nbuf

                @pl.when(nk < nchunk)
                def _issue():
                    pltpu.async_copy(tbl_h.at[ir_vm.at[nk]], bufr[b], gr[b])
                    pltpu.async_copy(tbl_h.at[ic_vm.at[nk]], bufc[b], gc[b])
            return carry

        lax.fori_loop(0, nchunk // nbuf, outer, 0)

    fn = pl.kernel(
        body,
        out_type=[jax.ShapeDtypeStruct((e_pad, h_dim), f32)] * 2,
        mesh=_mesh(),
        compiler_params=_SC_PARAMS,
        scratch_types=(
            [pltpu.VMEM((nchunk, CH), jnp.int32)] * 2
            + [pltpu.VMEM((CH, h_dim), f32)] * (2 * nbuf)
            + [pltpu.SemaphoreType.DMA] * (4 * nbuf)
        ),
    )
    return fn(tbl, row2, col2)


def _segsum_call(msg, row2, n_nodes):
    e_pad, h_dim = msg.shape
    nchunk = e_pad // (NW * CH)
    zr = n_nodes // NS
    nbuf = 2
    f32 = jnp.float32
    zeros = jnp.zeros((zr, h_dim), f32)

    def body(msg_h, row_h, zero_h, part_h, idx_vm, *bufs):
        mbuf = bufs[0:nbuf]
        lg = bufs[nbuf:2 * nbuf]
        la = bufs[2 * nbuf:3 * nbuf]
        cid = lax.axis_index("c")
        sid = lax.axis_index("s")
        wid = sid * NC + cid
        cb = wid * nchunk
        pltpu.sync_copy(row_h.at[pl.ds(cb, nchunk)], idx_vm)
        acc = bufs[3 * nbuf]
        pltpu.sync_copy(zero_h, acc.at[pl.ds(sid * zr, zr)])
        plsc.subcore_barrier()

        for b in range(nbuf):
            pltpu.async_copy(msg_h.at[pl.ds((cb + b) * CH, CH)], mbuf[b],
                             lg[b])

        def outer(i, carry):
            k0 = i * nbuf
            for b in range(nbuf):
                k = k0 + b
                pltpu.make_async_copy(msg_h.at[pl.ds((cb + k) * CH, CH)],
                                      mbuf[b], lg[b]).wait()
                pltpu.async_copy(mbuf[b], acc.at[idx_vm.at[k]], la[b],
                                 add=True)
                pltpu.make_async_copy(mbuf[b], acc.at[idx_vm.at[k]],
                                      la[b]).wait()
                nk = k + nbuf

                @pl.when(nk < nchunk)
                def _issue():
                    pltpu.async_copy(msg_h.at[pl.ds((cb + nk) * CH, CH)],
                                     mbuf[b], lg[b])
            return carry

        lax.fori_loop(0, nchunk // nbuf, outer, 0)
        plsc.subcore_barrier()
        pltpu.sync_copy(acc.at[pl.ds(sid * zr, zr)],
                        part_h.at[cid, pl.ds(sid * zr, zr)])

    fn = pl.kernel(
        body,
        out_type=jax.ShapeDtypeStruct((NC, n_nodes, h_dim), f32),
        mesh=_mesh(),
        compiler_params=_SC_PARAMS,
        scratch_types=(
            [pltpu.VMEM((nchunk, CH), jnp.int32)]
            + [pltpu.VMEM((CH, h_dim), f32)] * nbuf
            + [pltpu.SemaphoreType.DMA] * (2 * nbuf)
            + [pltpu.VMEM_SHARED((n_nodes, h_dim), f32)]
        ),
    )
    return fn(msg, row2, zeros)


def _edge_mlp_call(hr, hc, rad3, ea3, em3, w1a, w1b, wr, wa, b1, w2, b2,
                   last_t, last_b, attention):
    e_pad, h_dim = hr.shape
    g = e_pad // BE
    f32 = jnp.float32

    def body(hr_ref, hc_ref, rad_ref, ea_ref, em_ref, w1a_ref, w1b_ref,
             wr_ref, wa_ref, b1_ref, w2_ref, b2_ref, lt_ref, lb_ref, out_ref):
        z = (jnp.dot(hr_ref[...], w1a_ref[...], preferred_element_type=f32)
             + jnp.dot(hc_ref[...], w1b_ref[...], preferred_element_type=f32))
        z = (z + rad_ref[0, 0, :][:, None] * wr_ref[...]
             + ea_ref[0, 0, :][:, None] * wa_ref[...] + b1_ref[...])
        m = _silu(z)
        m = _silu(jnp.dot(m, w2_ref[...], preferred_element_type=f32)
                  + b2_ref[...])
        em = em_ref[0, 0, :]
        if attention:
            att = jax.nn.sigmoid(
                jnp.sum(m * lt_ref[...], axis=1, keepdims=True) + lb_ref[0, 0])
            out_ref[...] = m * att * em[:, None]
        else:
            out_ref[0, 0, :] = jnp.sum(m * lt_ref[...], axis=1) * em

    full = lambda shp: pl.BlockSpec(shp, lambda i: tuple(0 for _ in shp))
    in_specs = [
        pl.BlockSpec((BE, h_dim), lambda i: (i, 0)),
        pl.BlockSpec((BE, h_dim), lambda i: (i, 0)),
        pl.BlockSpec((1, 1, BE), lambda i: (i, 0, 0)),
        pl.BlockSpec((1, 1, BE), lambda i: (i, 0, 0)),
        pl.BlockSpec((1, 1, BE), lambda i: (i, 0, 0)),
        full((h_dim, h_dim)), full((h_dim, h_dim)),
        full((1, h_dim)), full((1, h_dim)), full((1, h_dim)),
        full((h_dim, h_dim)), full((1, h_dim)),
        full((1, h_dim)), full((1, 1)),
    ]
    if attention:
        out_specs = pl.BlockSpec((BE, h_dim), lambda i: (i, 0))
        out_shape = jax.ShapeDtypeStruct((e_pad, h_dim), f32)
    else:
        out_specs = pl.BlockSpec((1, 1, BE), lambda i: (i, 0, 0))
        out_shape = jax.ShapeDtypeStruct((g, 1, BE), f32)
    return pl.pallas_call(
        body, grid=(g,), in_specs=in_specs, out_specs=out_specs,
        out_shape=out_shape,
    )(hr, hc, rad3, ea3, em3, w1a, w1b, wr, wa, b1, w2, b2, last_t, last_b)


def _edge_equiv_call(hr, hc, rad3, ea3, em3, c03, c13, c23,
                     w1a, w1b, wr, wa, b1, w2, b2, w3t):
    e_pad, h_dim = hr.shape
    g = e_pad // BE
    f32 = jnp.float32

    def body(hr_ref, hc_ref, rad_ref, ea_ref, em_ref, c0_ref, c1_ref, c2_ref,
             w1a_ref, w1b_ref, wr_ref, wa_ref, b1_ref, w2_ref, b2_ref,
             w3_ref, out_ref):
        z = (jnp.dot(hr_ref[...], w1a_ref[...], preferred_element_type=f32)
             + jnp.dot(hc_ref[...], w1b_ref[...], preferred_element_type=f32))
        z = (z + rad_ref[0, 0, :][:, None] * wr_ref[...]
             + ea_ref[0, 0, :][:, None] * wa_ref[...] + b1_ref[...])
        m = _silu(z)
        m = _silu(jnp.dot(m, w2_ref[...], preferred_element_type=f32)
                  + b2_ref[...])
        phi = jnp.sum(m * w3_ref[...], axis=1) * em_ref[0, 0, :]
        cols = [(phi * c_ref[0, 0, :])[:, None]
                for c_ref in (c0_ref, c1_ref, c2_ref)]
        out_ref[...] = jnp.concatenate(
            cols + [jnp.zeros((BE, h_dim - 3), f32)], axis=1)

    full = lambda shp: pl.BlockSpec(shp, lambda i: tuple(0 for _ in shp))
    edge1 = lambda: pl.BlockSpec((1, 1, BE), lambda i: (i, 0, 0))
    in_specs = [
        pl.BlockSpec((BE, h_dim), lambda i: (i, 0)),
        pl.BlockSpec((BE, h_dim), lambda i: (i, 0)),
        edge1(), edge1(), edge1(), edge1(), edge1(), edge1(),
        full((h_dim, h_dim)), full((h_dim, h_dim)),
        full((1, h_dim)), full((1, h_dim)), full((1, h_dim)),
        full((h_dim, h_dim)), full((1, h_dim)), full((1, h_dim)),
    ]
    return pl.pallas_call(
        body, grid=(g,), in_specs=in_specs,
        out_specs=pl.BlockSpec((BE, h_dim), lambda i: (i, 0)),
        out_shape=jax.ShapeDtypeStruct((e_pad, h_dim), f32),
    )(hr, hc, rad3, ea3, em3, c03, c13, c23,
      w1a, w1b, wr, wa, b1, w2, b2, w3t)


def _node_mlp_call(h, part, nm3, w1a, w1b, b1, w2, b2):
    n_nodes, h_dim = h.shape
    bn = 512
    g = n_nodes // bn
    f32 = jnp.float32

    def body(h_ref, p_ref, nm_ref, w1a_ref, w1b_ref, b1_ref, w2_ref, b2_ref,
             out_ref):
        hv = h_ref[...]
        agg = (p_ref[0] + p_ref[1]) * (1.0 / NORM_FACTOR)
        z = (jnp.dot(hv, w1a_ref[...], preferred_element_type=f32)
             + jnp.dot(agg, w1b_ref[...], preferred_element_type=f32)
             + b1_ref[...])
        o = jnp.dot(_silu(z), w2_ref[...], preferred_element_type=f32) \
            + b2_ref[...]
        out_ref[...] = (hv + o) * nm_ref[0, 0, :][:, None]

    full = lambda shp: pl.BlockSpec(shp, lambda i: tuple(0 for _ in shp))
    return pl.pallas_call(
        body, grid=(g,),
        in_specs=[
            pl.BlockSpec((bn, h_dim), lambda i: (i, 0)),
            pl.BlockSpec((NC, bn, h_dim), lambda i: (0, i, 0)),
            pl.BlockSpec((1, 1, bn), lambda i: (i, 0, 0)),
            full((h_dim, h_dim)), full((h_dim, h_dim)), full((1, h_dim)),
            full((h_dim, h_dim)), full((1, h_dim)),
        ],
        out_specs=pl.BlockSpec((bn, h_dim), lambda i: (i, 0)),
        out_shape=jax.ShapeDtypeStruct((n_nodes, h_dim), f32),
    )(h, part, nm3, w1a, w1b, b1, w2, b2)


def _xupd_call(x16, px, nm3):
    n_nodes = x16.shape[0]
    hp = px.shape[2]
    bn = 512
    g = n_nodes // bn
    f32 = jnp.float32

    def body(x_ref, px_ref, nm_ref, out_ref):
        agg = (px_ref[0, :, :LANE16] + px_ref[1, :, :LANE16]) \
            * (1.0 / NORM_FACTOR)
        out_ref[...] = (x_ref[...] + agg) * nm_ref[0, 0, :][:, None]

    return pl.pallas_call(
        body, grid=(g,),
        in_specs=[
            pl.BlockSpec((bn, LANE16), lambda i: (i, 0)),
            pl.BlockSpec((NC, bn, hp), lambda i: (0, i, 0)),
            pl.BlockSpec((1, 1, bn), lambda i: (i, 0, 0)),
        ],
        out_specs=pl.BlockSpec((bn, LANE16), lambda i: (i, 0)),
        out_shape=jax.ShapeDtypeStruct((n_nodes, LANE16), f32),
    )(x16, px, nm3)


def kernel(h, x, edge_index, batch_size, node_mask, edge_mask, edge_attr,
           params):
    n_nodes, h_dim = h.shape
    e = edge_index.shape[1]
    quant = NW * CH * 8
    e_pad = ((e + quant - 1) // quant) * quant
    pad = e_pad - e
    n_pad = ((n_nodes + 511) // 512) * 512
    npad_rows = n_pad - n_nodes
    f32 = jnp.float32

    row1 = jnp.concatenate([edge_index[0], jnp.zeros((pad,), jnp.int32)])
    col1 = jnp.concatenate([edge_index[1], jnp.zeros((pad,), jnp.int32)])
    row2 = row1.reshape(-1, CH)
    col2 = col1.reshape(-1, CH)
    ea1 = jnp.concatenate([edge_attr[:, 0], jnp.zeros((pad,), f32)])
    em1 = jnp.concatenate([edge_mask[:, 0], jnp.zeros((pad,), f32)])

    ge = e_pad // BE
    ea3 = ea1.reshape(ge, 1, BE)
    em3 = em1.reshape(ge, 1, BE)
    nm1 = jnp.concatenate([node_mask[:, 0], jnp.zeros((npad_rows,), f32)])
    nm3 = nm1.reshape(n_pad // 512, 1, 512)
    xp = jnp.concatenate([x, jnp.zeros((npad_rows, x.shape[1]), f32)])

    rad1, c0, c1, c2 = _geom_call(xp[:, 0], xp[:, 1], xp[:, 2], row1, col1, n_pad, e_pad)
    rad3 = rad1.reshape(ge, 1, BE)

    hcur = jnp.concatenate([h, jnp.zeros((npad_rows, h_dim), f32)])
    for i in range(2):
        p = params['gcl%d' % i]
        w1a, w1b = p['eW1'][:h_dim], p['eW1'][h_dim:2 * h_dim]
        wr = p['eW1'][2 * h_dim:2 * h_dim + 1]
        wa = p['eW1'][2 * h_dim + 1:2 * h_dim + 2]
        hr, hc = _gather_call(hcur, row2, col2, e_pad)
        msg = _edge_mlp_call(hr, hc, rad3, ea3, em3, w1a, w1b, wr, wa,
                             p['eb1'].reshape(1, h_dim), p['eW2'],
                             p['eb2'].reshape(1, h_dim),
                             p['aW'].reshape(1, h_dim),
                             p['ab'].reshape(1, 1), attention=True)
        part = _segsum_call(msg, row2, n_pad)
        hcur = _node_mlp_call(hcur, part, nm3,
                              p['nW1'][:h_dim], p['nW1'][h_dim:],
                              p['nb1'].reshape(1, h_dim), p['nW2'],
                              p['nb2'].reshape(1, h_dim))

    p = params['equiv']
    w1a, w1b = p['cW1'][:h_dim], p['cW1'][h_dim:2 * h_dim]
    wr = p['cW1'][2 * h_dim:2 * h_dim + 1]
    wa = p['cW1'][2 * h_dim + 1:2 * h_dim + 2]
    hr, hc = _gather_call(hcur, row2, col2, e_pad)
    c03 = c0.reshape(ge, 1, BE)
    c13 = c1.reshape(ge, 1, BE)
    c23 = c2.reshape(ge, 1, BE)
    trans = _edge_equiv_call(hr, hc, rad3, ea3, em3, c03, c13, c23,
                               w1a, w1b, wr, wa,
                               p['cb1'].reshape(1, h_dim), p['cW2'],
                               p['cb2'].reshape(1, h_dim),
                               p['cW3'].reshape(1, h_dim))
    px = _segsum_call(trans, row2, n_pad)

    x16 = jnp.pad(xp, ((0, 0), (0, LANE16 - x.shape[1])))
    xo16 = _xupd_call(x16, px, nm3)
    x_new = xo16[:n_nodes, :x.shape[1]]
    return hcur[:n_nodes], x_new
